# Initial kernel scaffold; baseline (speedup 1.0000x reference)
import jax
import jax.numpy as jnp
from jax.experimental import pallas as pl


def kernel(x, edge_index, edge_type, node_type, type_emb, W_self0, W_rel0, b0, W_self1, W_rel1, b1):
    raise NotImplementedError("write your pallas kernel here")


# speedup vs baseline: 2.2848x; 2.2848x over previous
import functools

import jax
import jax.numpy as jnp
from jax import lax
from jax.experimental import pallas as pl
from jax.experimental.pallas import tpu as pltpu
from jax.experimental.pallas import tpu_sc as plsc

F = 32
GB = 128
NWORK = 32
NSUB = 16


def _sc_aggregate(h2, src_p, ridx_p, *, n_nodes, nch, r_rel, with_deg):
    rnp = -(-(r_rel * n_nodes + 1) // (NSUB * 128)) * (NSUB * 128)
    zr = rnp // NSUB
    nblk = src_p.shape[1]

    out_type = [jax.ShapeDtypeStruct((2, nch, rnp, F), jnp.float32)]
    if with_deg:
        out_type.append(jax.ShapeDtypeStruct((2, rnp, F), jnp.float32))

    mesh = plsc.VectorSubcoreMesh(core_axis_name="c", subcore_axis_name="s",
                                  num_cores=2, num_subcores=NSUB)

    def body(h2_hbm, src_hbm, ridx_hbm, acc_hbm, *rest):
        if with_deg:
            deg_hbm = rest[0]
            rest = rest[1:]
        acc_sh, src2d, ridx2d, gidx2d, gbuf, ones, zbuf, sem = rest
        cid = lax.axis_index("c")
        sid = lax.axis_index("s")
        wid = sid * 2 + cid

        pltpu.sync_copy(src_hbm.at[wid], src2d)
        pltpu.sync_copy(ridx_hbm.at[wid], ridx2d)

        def init_ones(i, _):
            ones[i, pl.ds(0, 16)] = jnp.ones((16,), jnp.float32)
            ones[i, pl.ds(16, 16)] = jnp.ones((16,), jnp.float32)
            return 0
        lax.fori_loop(0, GB, init_ones, 0)

        def init_z(i, _):
            zbuf[i, pl.ds(0, 16)] = jnp.zeros((16,), jnp.float32)
            zbuf[i, pl.ds(16, 16)] = jnp.zeros((16,), jnp.float32)
            return 0
        lax.fori_loop(0, GB, init_z, 0)

        my_rows = pl.ds(sid * zr, zr)

        def zero_my_stripe():
            def zz(k, _):
                pltpu.sync_copy(zbuf, acc_sh.at[pl.ds(sid * zr + k * GB, GB)])
                return 0
            lax.fori_loop(0, zr // GB, zz, 0)

        if with_deg:
            zero_my_stripe()
            plsc.subcore_barrier()

            def deg_blk(j, _):
                pltpu.sync_copy(ones, acc_sh.at[ridx2d.at[j]], add=True)
                return 0
            lax.fori_loop(0, nblk, deg_blk, 0)
            plsc.subcore_barrier()
            pltpu.sync_copy(acc_sh.at[my_rows], deg_hbm.at[cid, my_rows])
            plsc.subcore_barrier()

        def chunk(c, _):
            zero_my_stripe()
            def gi(j, _):
                for k in range(GB // 16):
                    v = src2d[j, pl.ds(k * 16, 16)]
                    gidx2d[j, pl.ds(k * 16, 16)] = v * nch + c
                return 0
            lax.fori_loop(0, nblk, gi, 0)
            plsc.subcore_barrier()

            def blk(j, _):
                pltpu.async_copy(h2_hbm.at[gidx2d.at[j]], gbuf, sem).wait()
                pltpu.sync_copy(gbuf, acc_sh.at[ridx2d.at[j]], add=True)
                return 0
            lax.fori_loop(0, nblk, blk, 0)
            plsc.subcore_barrier()
            pltpu.sync_copy(acc_sh.at[my_rows], acc_hbm.at[cid, c, my_rows])
            plsc.subcore_barrier()
            return 0
        lax.fori_loop(0, nch, chunk, 0)

    fn = pl.kernel(
        body,
        out_type=tuple(out_type),
        mesh=mesh,
        scratch_types=(
            pltpu.VMEM_SHARED((rnp, F), jnp.float32),
            pltpu.VMEM((nblk, GB), jnp.int32),
            pltpu.VMEM((nblk, GB), jnp.int32),
            pltpu.VMEM((nblk, GB), jnp.int32),
            pltpu.VMEM((GB, F), jnp.float32),
            pltpu.VMEM((GB, F), jnp.float32),
            pltpu.VMEM((GB, F), jnp.float32),
            pltpu.SemaphoreType.DMA,
        ),
        compiler_params=pltpu.CompilerParams(use_tc_tiling_on_sc=False),
    )
    return fn(h2, src_p, ridx_p)


def _concat_kernel(x, node_type3, type_emb, *, bn):
    n, d = x.shape
    nt, td = type_emb.shape
    nb = n // bn

    def body(x_ref, nt_ref, te_ref, o_ref):
        ids = nt_ref[0]
        temb = jnp.zeros((bn, td), jnp.float32)
        for t in range(nt):
            temb = jnp.where(ids == t, te_ref[t][None, :], temb)
        o_ref[...] = jnp.concatenate([x_ref[...], temb], axis=1)

    return pl.pallas_call(
        body,
        grid=(nb,),
        in_specs=[
            pl.BlockSpec((bn, d), lambda i: (i, 0)),
            pl.BlockSpec((1, bn, 1), lambda i: (i, 0, 0)),
            pl.BlockSpec((nt, td), lambda i: (0, 0)),
        ],
        out_specs=pl.BlockSpec((bn, d + td), lambda i: (i, 0)),
        out_shape=jax.ShapeDtypeStruct((n, d + td), jnp.float32),
    )(x, node_type3, type_emb)


def _repack_kernel(acc, deg, *, n_nodes, r_rel, bn):
    _, nch, rnp, f = acc.shape
    rn = r_rel * n_nodes
    nb = rn // bn
    cg = 128 // f
    assert nch % cg == 0

    def body(a_ref, d_ref, o_ref):
        d = d_ref[0, :, 0:1] + d_ref[1, :, 0:1]
        recip = 1.0 / jnp.maximum(d, 1.0)
        s = a_ref[0] + a_ref[1]
        o_ref[...] = jnp.concatenate([s[g] for g in range(cg)], axis=1) * recip

    return pl.pallas_call(
        body,
        grid=(nch // cg, nb),
        in_specs=[
            pl.BlockSpec((2, cg, bn, f), lambda c, i: (0, c, i, 0)),
            pl.BlockSpec((2, bn, f), lambda c, i: (0, i, 0)),
        ],
        out_specs=pl.BlockSpec((bn, cg * f), lambda c, i: (i, c)),
        out_shape=jax.ShapeDtypeStruct((rn, nch * f), jnp.float32),
    )(acc, deg)


def _layer_matmul(h, aggs, w_self, w_rel, b, *, bn):
    n, d_in = h.shape
    r_rel = w_rel.shape[0]
    d_out = w_self.shape[1]
    nb = n // bn

    def body(h_ref, a_ref, ws_ref, wr_ref, b_ref, o_ref):
        out = jnp.dot(h_ref[...], ws_ref[...],
                      preferred_element_type=jnp.float32) + b_ref[0][None, :]
        for r in range(r_rel):
            out += jnp.dot(a_ref[r], wr_ref[r],
                           preferred_element_type=jnp.float32)
        o_ref[...] = jnp.maximum(out, 0.0)

    return pl.pallas_call(
        body,
        grid=(nb,),
        in_specs=[
            pl.BlockSpec((bn, d_in), lambda i: (i, 0)),
            pl.BlockSpec((r_rel, bn, d_in), lambda i: (0, i, 0)),
            pl.BlockSpec((d_in, d_out), lambda i: (0, 0)),
            pl.BlockSpec((r_rel, d_in, d_out), lambda i: (0, 0, 0)),
            pl.BlockSpec((1, d_out), lambda i: (0, 0)),
        ],
        out_specs=pl.BlockSpec((bn, d_out), lambda i: (i, 0)),
        out_shape=jax.ShapeDtypeStruct((n, d_out), jnp.float32),
    )(h, aggs, w_self, w_rel, b)


def kernel(x, edge_index, edge_type, node_type, type_emb,
           W_self0, W_rel0, b0, W_self1, W_rel1, b1):
    n, d = x.shape
    e = edge_index.shape[1]
    nt, td = type_emb.shape
    r_rel = W_rel0.shape[0]
    d_in0 = d + td

    epw = -(-e // NWORK)
    epw = -(-epw // GB) * GB
    e_pad = epw * NWORK
    pad = e_pad - e
    src = jnp.concatenate([edge_index[0], jnp.zeros((pad,), jnp.int32)])
    ridx = edge_type * n + edge_index[1]
    ridx = jnp.concatenate([ridx, jnp.full((pad,), r_rel * n, jnp.int32)])
    src_p = src.reshape(NWORK, epw // GB, GB)
    ridx_p = ridx.reshape(NWORK, epw // GB, GB)
    node_type3 = node_type.reshape(n // 400, 400, 1)

    h0 = _concat_kernel(x, node_type3, type_emb, bn=400)
    nch0 = d_in0 // F
    acc0, deg = _sc_aggregate(h0.reshape(n * nch0, F), src_p, ridx_p,
                              n_nodes=n, nch=nch0, r_rel=r_rel, with_deg=True)
    agg0 = _repack_kernel(acc0, deg, n_nodes=n, r_rel=r_rel, bn=2000)
    h1 = _layer_matmul(h0, agg0.reshape(r_rel, n, d_in0),
                       W_self0, W_rel0, b0.reshape(1, -1), bn=400)

    nch1 = h1.shape[1] // F
    (acc1,) = _sc_aggregate(h1.reshape(n * nch1, F), src_p, ridx_p,
                            n_nodes=n, nch=nch1, r_rel=r_rel, with_deg=False)
    agg1 = _repack_kernel(acc1, deg, n_nodes=n, r_rel=r_rel, bn=2000)
    h2 = _layer_matmul(h1, agg1.reshape(r_rel, n, h1.shape[1]),
                       W_self1, W_rel1, b1.reshape(1, -1), bn=400)
    return h2


# speedup vs baseline: 2.7428x; 1.2005x over previous
import functools

import jax
import jax.numpy as jnp
from jax import lax
from jax.experimental import pallas as pl
from jax.experimental.pallas import tpu as pltpu
from jax.experimental.pallas import tpu_sc as plsc

F = 32
GB = 128
NWORK = 32
NBUF = 4
NSUB = 16


def _sc_aggregate(h2, src_p, ridx_p, *, n_nodes, nch, r_rel, with_deg):
    rnp = -(-(r_rel * n_nodes + 1) // (NSUB * 128)) * (NSUB * 128)
    zr = rnp // NSUB
    nblk = src_p.shape[1]

    out_type = [jax.ShapeDtypeStruct((2, nch, rnp, F), jnp.float32)]
    if with_deg:
        out_type.append(jax.ShapeDtypeStruct((2, rnp, F), jnp.float32))

    mesh = plsc.VectorSubcoreMesh(core_axis_name="c", subcore_axis_name="s",
                                  num_cores=2, num_subcores=NSUB)

    def body(h2_hbm, src_hbm, ridx_hbm, acc_hbm, *rest):
        if with_deg:
            deg_hbm = rest[0]
            rest = rest[1:]
        acc_sh, src2d, ridx2d, gidx2d, gbuf, ones, zbuf = rest[:7]
        gsem = rest[7:7 + NBUF]
        ssem = rest[7 + NBUF:7 + 2 * NBUF]
        cid = lax.axis_index("c")
        sid = lax.axis_index("s")
        wid = sid * 2 + cid

        pltpu.sync_copy(src_hbm.at[wid], src2d)
        pltpu.sync_copy(ridx_hbm.at[wid], ridx2d)

        def init_ones(i, _):
            ones[i, pl.ds(0, 16)] = jnp.ones((16,), jnp.float32)
            ones[i, pl.ds(16, 16)] = jnp.ones((16,), jnp.float32)
            return 0
        lax.fori_loop(0, GB, init_ones, 0)

        def init_z(i, _):
            zbuf[i, pl.ds(0, 16)] = jnp.zeros((16,), jnp.float32)
            zbuf[i, pl.ds(16, 16)] = jnp.zeros((16,), jnp.float32)
            return 0
        lax.fori_loop(0, GB, init_z, 0)

        my_rows = pl.ds(sid * zr, zr)

        def zero_my_stripe():
            def zz(k, _):
                pltpu.sync_copy(zbuf, acc_sh.at[pl.ds(sid * zr + k * GB, GB)])
                return 0
            lax.fori_loop(0, zr // GB, zz, 0)

        if with_deg:
            zero_my_stripe()
            plsc.subcore_barrier()

            def deg_blk(j, _):
                pltpu.sync_copy(ones, acc_sh.at[ridx2d.at[j]], add=True)
                return 0
            lax.fori_loop(0, nblk, deg_blk, 0)
            plsc.subcore_barrier()
            pltpu.sync_copy(acc_sh.at[my_rows], deg_hbm.at[cid, my_rows])
            plsc.subcore_barrier()

        def chunk(c, _):
            zero_my_stripe()
            def gi(j, _):
                for k in range(GB // 16):
                    v = src2d[j, pl.ds(k * 16, 16)]
                    gidx2d[j, pl.ds(k * 16, 16)] = v * nch + c
                return 0
            lax.fori_loop(0, nblk, gi, 0)
            plsc.subcore_barrier()

            for b in range(NBUF):
                pltpu.async_copy(h2_hbm.at[gidx2d.at[b]], gbuf.at[b], gsem[b])

            def ring(g, _):
                for b in range(NBUF):
                    j = g * NBUF + b
                    pltpu.make_async_copy(h2_hbm.at[gidx2d.at[b]],
                                          gbuf.at[b], gsem[b]).wait()
                    pltpu.async_copy(gbuf.at[b], acc_sh.at[ridx2d.at[j]],
                                     ssem[b], add=True)
                    pltpu.make_async_copy(gbuf.at[b], acc_sh.at[ridx2d.at[j]],
                                          ssem[b]).wait()

                    @pl.when(j + NBUF < nblk)
                    def _():
                        pltpu.async_copy(h2_hbm.at[gidx2d.at[j + NBUF]],
                                         gbuf.at[b], gsem[b])
                return 0
            lax.fori_loop(0, nblk // NBUF, ring, 0)
            plsc.subcore_barrier()
            pltpu.sync_copy(acc_sh.at[my_rows], acc_hbm.at[cid, c, my_rows])
            plsc.subcore_barrier()
            return 0
        lax.fori_loop(0, nch, chunk, 0)

    fn = pl.kernel(
        body,
        out_type=tuple(out_type),
        mesh=mesh,
        scratch_types=(
            pltpu.VMEM_SHARED((rnp, F), jnp.float32),
            pltpu.VMEM((nblk, GB), jnp.int32),
            pltpu.VMEM((nblk, GB), jnp.int32),
            pltpu.VMEM((nblk, GB), jnp.int32),
            pltpu.VMEM((NBUF, GB, F), jnp.float32),
            pltpu.VMEM((GB, F), jnp.float32),
            pltpu.VMEM((GB, F), jnp.float32),
        ) + (pltpu.SemaphoreType.DMA,) * (2 * NBUF),
        compiler_params=pltpu.CompilerParams(use_tc_tiling_on_sc=False),
    )
    return fn(h2, src_p, ridx_p)


def _concat_kernel(x, node_type3, type_emb, *, bn):
    n, d = x.shape
    nt, td = type_emb.shape
    nb = n // bn

    def body(x_ref, nt_ref, te_ref, o_ref):
        ids = nt_ref[0]
        temb = jnp.zeros((bn, td), jnp.float32)
        for t in range(nt):
            temb = jnp.where(ids == t, te_ref[t][None, :], temb)
        o_ref[...] = jnp.concatenate([x_ref[...], temb], axis=1)

    return pl.pallas_call(
        body,
        grid=(nb,),
        in_specs=[
            pl.BlockSpec((bn, d), lambda i: (i, 0)),
            pl.BlockSpec((1, bn, 1), lambda i: (i, 0, 0)),
            pl.BlockSpec((nt, td), lambda i: (0, 0)),
        ],
        out_specs=pl.BlockSpec((bn, d + td), lambda i: (i, 0)),
        out_shape=jax.ShapeDtypeStruct((n, d + td), jnp.float32),
    )(x, node_type3, type_emb)


def _repack_kernel(acc, deg, *, n_nodes, r_rel, bn):
    _, nch, rnp, f = acc.shape
    rn = r_rel * n_nodes
    nb = rn // bn
    cg = 128 // f
    assert nch % cg == 0

    def body(a_ref, d_ref, o_ref):
        d = d_ref[0, :, 0:1] + d_ref[1, :, 0:1]
        recip = 1.0 / jnp.maximum(d, 1.0)
        s = a_ref[0] + a_ref[1]
        o_ref[...] = jnp.concatenate([s[g] for g in range(cg)], axis=1) * recip

    return pl.pallas_call(
        body,
        grid=(nch // cg, nb),
        in_specs=[
            pl.BlockSpec((2, cg, bn, f), lambda c, i: (0, c, i, 0)),
            pl.BlockSpec((2, bn, f), lambda c, i: (0, i, 0)),
        ],
        out_specs=pl.BlockSpec((bn, cg * f), lambda c, i: (i, c)),
        out_shape=jax.ShapeDtypeStruct((rn, nch * f), jnp.float32),
    )(acc, deg)


def _layer_matmul(h, aggs, w_self, w_rel, b, *, bn):
    n, d_in = h.shape
    r_rel = w_rel.shape[0]
    d_out = w_self.shape[1]
    nb = n // bn

    def body(h_ref, a_ref, ws_ref, wr_ref, b_ref, o_ref):
        out = jnp.dot(h_ref[...], ws_ref[...],
                      preferred_element_type=jnp.float32) + b_ref[0][None, :]
        for r in range(r_rel):
            out += jnp.dot(a_ref[r], wr_ref[r],
                           preferred_element_type=jnp.float32)
        o_ref[...] = jnp.maximum(out, 0.0)

    return pl.pallas_call(
        body,
        grid=(nb,),
        in_specs=[
            pl.BlockSpec((bn, d_in), lambda i: (i, 0)),
            pl.BlockSpec((r_rel, bn, d_in), lambda i: (0, i, 0)),
            pl.BlockSpec((d_in, d_out), lambda i: (0, 0)),
            pl.BlockSpec((r_rel, d_in, d_out), lambda i: (0, 0, 0)),
            pl.BlockSpec((1, d_out), lambda i: (0, 0)),
        ],
        out_specs=pl.BlockSpec((bn, d_out), lambda i: (i, 0)),
        out_shape=jax.ShapeDtypeStruct((n, d_out), jnp.float32),
    )(h, aggs, w_self, w_rel, b)


def kernel(x, edge_index, edge_type, node_type, type_emb,
           W_self0, W_rel0, b0, W_self1, W_rel1, b1):
    n, d = x.shape
    e = edge_index.shape[1]
    nt, td = type_emb.shape
    r_rel = W_rel0.shape[0]
    d_in0 = d + td

    epw = -(-e // NWORK)
    epw = -(-epw // GB) * GB
    e_pad = epw * NWORK
    pad = e_pad - e
    src = jnp.concatenate([edge_index[0], jnp.zeros((pad,), jnp.int32)])
    ridx = edge_type * n + edge_index[1]
    ridx = jnp.concatenate([ridx, jnp.full((pad,), r_rel * n, jnp.int32)])
    src_p = src.reshape(NWORK, epw // GB, GB)
    ridx_p = ridx.reshape(NWORK, epw // GB, GB)
    node_type3 = node_type.reshape(n // 400, 400, 1)

    h0 = _concat_kernel(x, node_type3, type_emb, bn=400)
    nch0 = d_in0 // F
    acc0, deg = _sc_aggregate(h0.reshape(n * nch0, F), src_p, ridx_p,
                              n_nodes=n, nch=nch0, r_rel=r_rel, with_deg=True)
    agg0 = _repack_kernel(acc0, deg, n_nodes=n, r_rel=r_rel, bn=2000)
    h1 = _layer_matmul(h0, agg0.reshape(r_rel, n, d_in0),
                       W_self0, W_rel0, b0.reshape(1, -1), bn=400)

    nch1 = h1.shape[1] // F
    (acc1,) = _sc_aggregate(h1.reshape(n * nch1, F), src_p, ridx_p,
                            n_nodes=n, nch=nch1, r_rel=r_rel, with_deg=False)
    agg1 = _repack_kernel(acc1, deg, n_nodes=n, r_rel=r_rel, bn=2000)
    h2 = _layer_matmul(h1, agg1.reshape(r_rel, n, h1.shape[1]),
                       W_self1, W_rel1, b1.reshape(1, -1), bn=400)
    return h2
